# Initial kernel scaffold; baseline (speedup 1.0000x reference)
import jax
import jax.numpy as jnp
from jax.experimental import pallas as pl


def kernel(x, edge_index, edge_attr, W1, b1, W2, b2, W3, b3):
    raise NotImplementedError("write your pallas kernel here")


# speedup vs baseline: 6.4008x; 6.4008x over previous
import functools

import jax
import jax.numpy as jnp
from jax import lax
from jax.experimental import pallas as pl
from jax.experimental.pallas import tpu as pltpu
from jax.experimental.pallas import tpu_sc as plsc

NC = 2
NS = 16
K = 128


def _deg_kernel(NP, EP):
    T = EP // (NC * NS)
    G = T // K
    NR = NP // NS
    ZR = 128
    mesh = plsc.VectorSubcoreMesh(core_axis_name="c", subcore_axis_name="s")

    @functools.partial(
        pl.kernel,
        mesh=mesh,
        out_type=jax.ShapeDtypeStruct((NC, NP, 128), jnp.float32),
        scratch_types=[
            pltpu.VMEM((K,), jnp.int32),
            pltpu.VMEM((K,), jnp.float32),
            pltpu.VMEM((K, 128), jnp.float32),
            pltpu.VMEM_SHARED((NP, 128), jnp.float32),
        ],
    )
    def deg_k(col_hbm, ew_hbm, z_hbm, out_hbm, colv, ewv, rows, acc):
        c = lax.axis_index("c")
        s = lax.axis_index("s")

        pltpu.sync_copy(z_hbm, rows)
        for i in range(NR // ZR):
            pltpu.sync_copy(z_hbm, acc.at[pl.ds(s * NR + i * ZR, ZR)])
        plsc.subcore_barrier()

        tbase = (c * NS + s) * T

        def chunk(g, _):
            base = tbase + g * K
            pltpu.sync_copy(col_hbm.at[pl.ds(base, K)], colv)
            pltpu.sync_copy(ew_hbm.at[pl.ds(base, K)], ewv)

            def grp(g2, _):
                ews16 = ewv[pl.ds(g2 * 16, 16)]
                for l in range(16):
                    j = g2 * 16 + l
                    v = rows[j, pl.ds(0, 16)]
                    rows[j, pl.ds(0, 16)] = v * 0.0 + ews16[l]
                return 0
            lax.fori_loop(0, K // 16, grp, 0)

            pltpu.sync_copy(rows, acc.at[colv], add=True)
            return 0
        lax.fori_loop(0, G, chunk, 0)

        plsc.subcore_barrier()
        pltpu.sync_copy(acc.at[pl.ds(s * NR, NR)],
                        out_hbm.at[c, pl.ds(s * NR, NR)])

    return deg_k


def _agg_kernel(N, NP, EP, Fh, split_features):
    T = EP // NS if split_features else EP // (NC * NS)
    G = T // K
    NR = NP // NS
    ZR = 128
    FB = Fh // 16
    mesh = plsc.VectorSubcoreMesh(core_axis_name="c", subcore_axis_name="s")

    @functools.partial(
        pl.kernel,
        mesh=mesh,
        out_type=jax.ShapeDtypeStruct((NC, NP, Fh), jnp.float32),
        scratch_types=[
            pltpu.VMEM((K,), jnp.int32),
            pltpu.VMEM((K,), jnp.int32),
            pltpu.VMEM((K,), jnp.float32),
            pltpu.VMEM((K, Fh), jnp.float32),
            pltpu.VMEM_SHARED((NP, Fh), jnp.float32),
            pltpu.SemaphoreType.DMA,
        ],
    )
    def agg_k(h_hbm, row_hbm, col_hbm, ew_hbm, z_hbm, out_hbm,
              rowv, colv, ewv, rows, acc, sem):
        c = lax.axis_index("c")
        s = lax.axis_index("s")

        for i in range(NR // ZR):
            pltpu.sync_copy(z_hbm, acc.at[pl.ds(s * NR + i * ZR, ZR)])
        plsc.subcore_barrier()

        if split_features:
            roff = c * N
            tbase = s * T
        else:
            roff = None
            tbase = (c * NS + s) * T

        def chunk(g, _):
            base = tbase + g * K
            pltpu.sync_copy(row_hbm.at[pl.ds(base, K)], rowv)
            pltpu.sync_copy(col_hbm.at[pl.ds(base, K)], colv)
            pltpu.sync_copy(ew_hbm.at[pl.ds(base, K)], ewv)

            if roff is not None:
                def adj(j, _):
                    v = rowv[pl.ds(j * 16, 16)]
                    rowv[pl.ds(j * 16, 16)] = v + roff
                    return 0
                lax.fori_loop(0, K // 16, adj, 0)

            pltpu.async_copy(h_hbm.at[rowv], rows, sem).wait()

            def scale(g, _):
                ews16 = ewv[pl.ds(g * 16, 16)]
                for l in range(16):
                    j = g * 16 + l
                    e = ews16[l]
                    for f in range(FB):
                        v = rows[j, pl.ds(f * 16, 16)]
                        rows[j, pl.ds(f * 16, 16)] = v * e
                return 0
            lax.fori_loop(0, K // 16, scale, 0)

            pltpu.sync_copy(rows, acc.at[colv], add=True)
            return 0
        lax.fori_loop(0, G, chunk, 0)

        plsc.subcore_barrier()
        pltpu.sync_copy(acc.at[pl.ds(s * NR, NR)],
                        out_hbm.at[c, pl.ds(s * NR, NR)])

    return agg_k


_DOT = functools.partial(
    jax.lax.dot_general,
    dimension_numbers=(((1,), (0,)), ((), ())),
    precision=jax.lax.Precision.HIGHEST,
    preferred_element_type=jnp.float32,
)


def _dinv_kernel(N, R):
    def body(d_ref, out_ref):
        deg = d_ref[0, :, 0:1] + d_ref[1, :, 0:1]
        safe = jnp.where(deg > 0, deg, 1.0)
        out_ref[...] = jnp.where(deg > 0, lax.rsqrt(safe), 0.0)

    return pl.pallas_call(
        body,
        grid=(N // R,),
        in_specs=[pl.BlockSpec((NC, R, 128), lambda i: (0, i, 0))],
        out_specs=pl.BlockSpec((R, 1), lambda i: (i, 0)),
        out_shape=jax.ShapeDtypeStruct((N, 1), jnp.float32),
    )


def _mm_first_kernel(N, F_in, F_out, R):
    Fh = F_out // 2

    def body(x_ref, w_ref, dinv_ref, out_ref):
        h = _DOT(x_ref[...], w_ref[...]) * dinv_ref[...]
        out_ref[0] = h[:, :Fh]
        out_ref[1] = h[:, Fh:]

    return pl.pallas_call(
        body,
        grid=(N // R,),
        in_specs=[
            pl.BlockSpec((R, F_in), lambda i: (i, 0)),
            pl.BlockSpec((F_in, F_out), lambda i: (0, 0)),
            pl.BlockSpec((R, 1), lambda i: (i, 0)),
        ],
        out_specs=pl.BlockSpec((NC, R, Fh), lambda i: (0, i, 0)),
        out_shape=jax.ShapeDtypeStruct((NC, N, Fh), jnp.float32),
    )


def _mm_mid_kernel(N, NP, F_in, F_out, R, split_out):
    Fih = F_in // 2
    Fh = F_out // 2

    def body(s_ref, b_ref, dinv_ref, w_ref, out_ref):
        dinv = dinv_ref[...]
        xa = jnp.maximum(s_ref[0] * dinv + b_ref[:, :Fih], 0.0)
        xb = jnp.maximum(s_ref[1] * dinv + b_ref[:, Fih:], 0.0)
        acc = _DOT(xa, w_ref[:Fih, :]) + _DOT(xb, w_ref[Fih:, :])
        h = acc * dinv
        if split_out:
            out_ref[0] = h[:, :Fh]
            out_ref[1] = h[:, Fh:]
        else:
            out_ref[...] = h

    if split_out:
        out_specs = pl.BlockSpec((NC, R, Fh), lambda i: (0, i, 0))
        out_shape = jax.ShapeDtypeStruct((NC, N, Fh), jnp.float32)
    else:
        out_specs = pl.BlockSpec((R, F_out), lambda i: (i, 0))
        out_shape = jax.ShapeDtypeStruct((N, F_out), jnp.float32)

    return pl.pallas_call(
        body,
        grid=(N // R,),
        in_specs=[
            pl.BlockSpec((NC, R, Fih), lambda i: (0, i, 0)),
            pl.BlockSpec((1, F_in), lambda i: (0, 0)),
            pl.BlockSpec((R, 1), lambda i: (i, 0)),
            pl.BlockSpec((F_in, F_out), lambda i: (0, 0)),
        ],
        out_specs=out_specs,
        out_shape=out_shape,
    )


def _final_kernel(N, F, R):
    def body(s_ref, b_ref, dinv_ref, h_ref, ls_ref):
        dinv = dinv_ref[...]
        z = (s_ref[0] + s_ref[1]) * dinv + b_ref[...]
        m = jnp.max(z, axis=1, keepdims=True)
        lse = jnp.log(jnp.sum(jnp.exp(z - m), axis=1, keepdims=True)) + m
        h_ref[...] = z
        ls_ref[...] = z - lse

    return pl.pallas_call(
        body,
        grid=(N // R,),
        in_specs=[
            pl.BlockSpec((NC, R, F), lambda i: (0, i, 0)),
            pl.BlockSpec((1, F), lambda i: (0, 0)),
            pl.BlockSpec((R, 1), lambda i: (i, 0)),
        ],
        out_specs=[
            pl.BlockSpec((R, F), lambda i: (i, 0)),
            pl.BlockSpec((R, F), lambda i: (i, 0)),
        ],
        out_shape=(
            jax.ShapeDtypeStruct((N, F), jnp.float32),
            jax.ShapeDtypeStruct((N, F), jnp.float32),
        ),
    )


def kernel(x, edge_index, edge_attr, W1, b1, W2, b2, W3, b3):
    N, F0 = x.shape
    E = edge_index.shape[1]
    F1, F2, F3 = W1.shape[1], W2.shape[1], W3.shape[1]
    R = 1000

    loop = jnp.arange(N, dtype=edge_index.dtype)
    row_e = jnp.concatenate([edge_index[0], loop])
    col_e = jnp.concatenate([edge_index[1], loop])
    ew_e = jnp.concatenate([edge_attr, jnp.ones((N,), x.dtype)])
    EE = E + N
    align = NC * NS * K
    EP = ((EE + align - 1) // align) * align
    NP = ((N + NS * K - 1) // (NS * K)) * (NS * K)
    pad = EP - EE
    pidx = (jnp.arange(pad, dtype=edge_index.dtype) * 97) % N
    row_p = jnp.concatenate([row_e, pidx])
    col_p = jnp.concatenate([col_e, pidx])
    ew_p = jnp.concatenate([ew_e, jnp.zeros((pad,), x.dtype)])

    z128 = jnp.zeros((128, 128), jnp.float32)

    degacc = _deg_kernel(NP, EP)(col_p, ew_p, z128)
    dinv = _dinv_kernel(N, R)(degacc)

    b1r = b1.reshape(1, F1)
    b2r = b2.reshape(1, F2)
    b3r = b3.reshape(1, F3)

    H1 = _mm_first_kernel(N, F0, F1, R)(x, W1, dinv)
    S1 = _agg_kernel(N, NP, EP, F1 // 2, True)(H1.reshape(NC * N, F1 // 2),
                                               row_p, col_p, ew_p, z128)
    H2 = _mm_mid_kernel(N, NP, F1, F2, R, True)(S1, b1r, dinv, W2)
    S2 = _agg_kernel(N, NP, EP, F2 // 2, True)(H2.reshape(NC * N, F2 // 2),
                                               row_p, col_p, ew_p, z128)
    H3 = _mm_mid_kernel(N, NP, F2, F3, R, False)(S2, b2r, dinv, W3)
    S3 = _agg_kernel(N, NP, EP, F3, False)(H3, row_p, col_p, ew_p, z128)
    h, ls = _final_kernel(N, F3, R)(S3, b3r, dinv)
    return (h, ls)


# speedup vs baseline: 10.8821x; 1.7001x over previous
import functools

import jax
import jax.numpy as jnp
from jax import lax
from jax.experimental import pallas as pl
from jax.experimental.pallas import tpu as pltpu
from jax.experimental.pallas import tpu_sc as plsc

NC = 2
NS = 16
K = 128


def _deg_kernel(NP, EP):
    T = EP // (NC * NS)
    G = T // K
    NR = NP // NS
    ZR = 128
    mesh = plsc.VectorSubcoreMesh(core_axis_name="c", subcore_axis_name="s")

    @functools.partial(
        pl.kernel,
        mesh=mesh,
        out_type=jax.ShapeDtypeStruct((NC, NP, 128), jnp.float32),
        scratch_types=[
            pltpu.VMEM((2, 4, K), jnp.int32),
            pltpu.VMEM((2, K, 128), jnp.float32),
            pltpu.VMEM_SHARED((NP, 128), jnp.float32),
            pltpu.SemaphoreType.DMA,
            pltpu.SemaphoreType.DMA,
        ],
    )
    def deg_k(pk_hbm, z_hbm, out_hbm, pk, rows, acc, s0, s1):
        c = lax.axis_index("c")
        s = lax.axis_index("s")
        ssem = (s0, s1)

        pltpu.sync_copy(z_hbm, rows.at[0])
        pltpu.sync_copy(z_hbm, rows.at[1])
        for i in range(NR // ZR):
            pltpu.sync_copy(z_hbm, acc.at[pl.ds(s * NR + i * ZR, ZR)])
        plsc.subcore_barrier()

        cbase = (c * NS + s) * (T // K)

        def do_chunk(g, b):
            ewr = pk.at[b]

            def grp(g2, _):
                ei = ewr[3, pl.ds(g2 * 16, 16)]
                ews16 = lax.bitcast_convert_type(ei, jnp.float32)
                for l in range(16):
                    j = g2 * 16 + l
                    v = rows[b, j, pl.ds(0, 16)]
                    rows[b, j, pl.ds(0, 16)] = v * 0.0 + ews16[l]
                return 0
            lax.fori_loop(0, K // 16, grp, 0)
            pltpu.async_copy(rows.at[b], acc.at[pk.at[b].at[2]], ssem[b],
                             add=True)

        def wait_scatter(b):
            pltpu.make_async_copy(z_hbm, rows.at[b], ssem[b]).wait()

        pltpu.sync_copy(pk_hbm.at[cbase], pk.at[0])

        def pair(g2, _):
            g = g2 * 2
            @pl.when(g2 > 0)
            def _():
                wait_scatter(1)
            pltpu.sync_copy(pk_hbm.at[cbase + g + 1], pk.at[1])
            do_chunk(g, 0)

            @pl.when(g2 < G // 2 - 1)
            def _():
                wait_scatter(0)
                pltpu.sync_copy(pk_hbm.at[cbase + g + 2], pk.at[0])
            do_chunk(g + 1, 1)
            return 0
        lax.fori_loop(0, G // 2, pair, 0)
        wait_scatter(0)
        wait_scatter(1)

        plsc.subcore_barrier()
        pltpu.sync_copy(acc.at[pl.ds(s * NR, NR)],
                        out_hbm.at[c, pl.ds(s * NR, NR)])

    return deg_k


def _agg_kernel(N, NP, EP, Fh, split_features):
    T = EP // NS if split_features else EP // (NC * NS)
    G = T // K
    NR = NP // NS
    ZR = 128
    FB = Fh // 16
    mesh = plsc.VectorSubcoreMesh(core_axis_name="c", subcore_axis_name="s")

    @functools.partial(
        pl.kernel,
        mesh=mesh,
        out_type=jax.ShapeDtypeStruct((NC, NP, Fh), jnp.float32),
        scratch_types=[
            pltpu.VMEM((2, 4, K), jnp.int32),
            pltpu.VMEM((2, K, Fh), jnp.float32),
            pltpu.VMEM_SHARED((NP, Fh), jnp.float32),
            pltpu.SemaphoreType.DMA,
            pltpu.SemaphoreType.DMA,
            pltpu.SemaphoreType.DMA,
            pltpu.SemaphoreType.DMA,
        ],
    )
    def agg_k(h_hbm, pk_hbm, z_hbm, out_hbm, pk, rows, acc, g0, g1, t0, t1):
        c = lax.axis_index("c")
        s = lax.axis_index("s")
        gsem = (g0, g1)
        ssem = (t0, t1)

        for i in range(NR // ZR):
            pltpu.sync_copy(z_hbm, acc.at[pl.ds(s * NR + i * ZR, ZR)])
        plsc.subcore_barrier()

        if split_features:
            cbase = s * (T // K)
        else:
            cbase = (c * NS + s) * (T // K)

        def row_idx(b):
            if split_features:
                return pk.at[b].at[c]
            return pk.at[b].at[0]

        def issue(ci, b):
            pltpu.sync_copy(pk_hbm.at[ci], pk.at[b])
            pltpu.async_copy(h_hbm.at[row_idx(b)], rows.at[b], gsem[b])

        def process(b):
            pltpu.make_async_copy(h_hbm.at[pl.ds(0, K)], rows.at[b],
                                  gsem[b]).wait()

            def scale(g2, _):
                ei = pk[b, 3, pl.ds(g2 * 16, 16)]
                ews16 = lax.bitcast_convert_type(ei, jnp.float32)
                for l in range(16):
                    j = g2 * 16 + l
                    e = ews16[l]
                    for f in range(FB):
                        v = rows[b, j, pl.ds(f * 16, 16)]
                        rows[b, j, pl.ds(f * 16, 16)] = v * e
                return 0
            lax.fori_loop(0, K // 16, scale, 0)
            pltpu.async_copy(rows.at[b], acc.at[pk.at[b].at[2]], ssem[b],
                             add=True)

        def wait_scatter(b):
            pltpu.make_async_copy(h_hbm.at[pl.ds(0, K)], rows.at[b],
                                  ssem[b]).wait()

        issue(cbase, 0)

        def pair(g2, _):
            g = g2 * 2

            @pl.when(g2 > 0)
            def _():
                wait_scatter(1)
            issue(cbase + g + 1, 1)
            process(0)

            @pl.when(g2 < G // 2 - 1)
            def _():
                wait_scatter(0)
                issue(cbase + g + 2, 0)
            process(1)
            return 0
        lax.fori_loop(0, G // 2, pair, 0)
        wait_scatter(0)
        wait_scatter(1)

        plsc.subcore_barrier()
        pltpu.sync_copy(acc.at[pl.ds(s * NR, NR)],
                        out_hbm.at[c, pl.ds(s * NR, NR)])

    return agg_k


_DOT = functools.partial(
    jax.lax.dot_general,
    dimension_numbers=(((1,), (0,)), ((), ())),
    precision=jax.lax.Precision.HIGHEST,
    preferred_element_type=jnp.float32,
)


def _dinv_kernel(N, R):
    def body(d_ref, out_ref):
        deg = d_ref[0, :, 0:1] + d_ref[1, :, 0:1]
        safe = jnp.where(deg > 0, deg, 1.0)
        out_ref[...] = jnp.where(deg > 0, lax.rsqrt(safe), 0.0)

    return pl.pallas_call(
        body,
        grid=(N // R,),
        in_specs=[pl.BlockSpec((NC, R, 128), lambda i: (0, i, 0))],
        out_specs=pl.BlockSpec((R, 1), lambda i: (i, 0)),
        out_shape=jax.ShapeDtypeStruct((N, 1), jnp.float32),
    )


def _mm_first_kernel(N, F_in, F_out, R):
    Fh = F_out // 2

    def body(x_ref, w_ref, dinv_ref, out_ref):
        h = _DOT(x_ref[...], w_ref[...]) * dinv_ref[...]
        out_ref[0] = h[:, :Fh]
        out_ref[1] = h[:, Fh:]

    return pl.pallas_call(
        body,
        grid=(N // R,),
        in_specs=[
            pl.BlockSpec((R, F_in), lambda i: (i, 0)),
            pl.BlockSpec((F_in, F_out), lambda i: (0, 0)),
            pl.BlockSpec((R, 1), lambda i: (i, 0)),
        ],
        out_specs=pl.BlockSpec((NC, R, Fh), lambda i: (0, i, 0)),
        out_shape=jax.ShapeDtypeStruct((NC, N, Fh), jnp.float32),
    )


def _mm_mid_kernel(N, NP, F_in, F_out, R, split_out):
    Fih = F_in // 2
    Fh = F_out // 2

    def body(s_ref, b_ref, dinv_ref, w_ref, out_ref):
        dinv = dinv_ref[...]
        xa = jnp.maximum(s_ref[0] * dinv + b_ref[:, :Fih], 0.0)
        xb = jnp.maximum(s_ref[1] * dinv + b_ref[:, Fih:], 0.0)
        acc = _DOT(xa, w_ref[:Fih, :]) + _DOT(xb, w_ref[Fih:, :])
        h = acc * dinv
        if split_out:
            out_ref[0] = h[:, :Fh]
            out_ref[1] = h[:, Fh:]
        else:
            out_ref[...] = h

    if split_out:
        out_specs = pl.BlockSpec((NC, R, Fh), lambda i: (0, i, 0))
        out_shape = jax.ShapeDtypeStruct((NC, N, Fh), jnp.float32)
    else:
        out_specs = pl.BlockSpec((R, F_out), lambda i: (i, 0))
        out_shape = jax.ShapeDtypeStruct((N, F_out), jnp.float32)

    return pl.pallas_call(
        body,
        grid=(N // R,),
        in_specs=[
            pl.BlockSpec((NC, R, Fih), lambda i: (0, i, 0)),
            pl.BlockSpec((1, F_in), lambda i: (0, 0)),
            pl.BlockSpec((R, 1), lambda i: (i, 0)),
            pl.BlockSpec((F_in, F_out), lambda i: (0, 0)),
        ],
        out_specs=out_specs,
        out_shape=out_shape,
    )


def _final_kernel(N, F, R):
    def body(s_ref, b_ref, dinv_ref, h_ref, ls_ref):
        dinv = dinv_ref[...]
        z = (s_ref[0] + s_ref[1]) * dinv + b_ref[...]
        m = jnp.max(z, axis=1, keepdims=True)
        lse = jnp.log(jnp.sum(jnp.exp(z - m), axis=1, keepdims=True)) + m
        h_ref[...] = z
        ls_ref[...] = z - lse

    return pl.pallas_call(
        body,
        grid=(N // R,),
        in_specs=[
            pl.BlockSpec((NC, R, F), lambda i: (0, i, 0)),
            pl.BlockSpec((1, F), lambda i: (0, 0)),
            pl.BlockSpec((R, 1), lambda i: (i, 0)),
        ],
        out_specs=[
            pl.BlockSpec((R, F), lambda i: (i, 0)),
            pl.BlockSpec((R, F), lambda i: (i, 0)),
        ],
        out_shape=(
            jax.ShapeDtypeStruct((N, F), jnp.float32),
            jax.ShapeDtypeStruct((N, F), jnp.float32),
        ),
    )


def kernel(x, edge_index, edge_attr, W1, b1, W2, b2, W3, b3):
    N, F0 = x.shape
    E = edge_index.shape[1]
    F1, F2, F3 = W1.shape[1], W2.shape[1], W3.shape[1]
    R = 1000

    loop = jnp.arange(N, dtype=edge_index.dtype)
    row_e = jnp.concatenate([edge_index[0], loop])
    col_e = jnp.concatenate([edge_index[1], loop])
    ew_e = jnp.concatenate([edge_attr, jnp.ones((N,), x.dtype)])
    EE = E + N
    align = NC * NS * K
    EP = ((EE + align - 1) // align) * align
    NP = ((N + NS * K - 1) // (NS * K)) * (NS * K)
    pad = EP - EE
    pidx = (jnp.arange(pad, dtype=edge_index.dtype) * 97) % N
    row_p = jnp.concatenate([row_e, pidx])
    col_p = jnp.concatenate([col_e, pidx])
    ew_p = jnp.concatenate([ew_e, jnp.zeros((pad,), x.dtype)])

    z128 = jnp.zeros((128, 128), jnp.float32)

    rowm = row_p.reshape(EP // K, K)
    colm = col_p.reshape(EP // K, K)
    ewb = jax.lax.bitcast_convert_type(ew_p, jnp.int32).reshape(EP // K, K)
    pk = jnp.stack([rowm, rowm + N, colm, ewb], axis=1)

    degacc = _deg_kernel(NP, EP)(pk, z128)
    dinv = _dinv_kernel(N, R)(degacc)

    b1r = b1.reshape(1, F1)
    b2r = b2.reshape(1, F2)
    b3r = b3.reshape(1, F3)

    H1 = _mm_first_kernel(N, F0, F1, R)(x, W1, dinv)
    S1 = _agg_kernel(N, NP, EP, F1 // 2, True)(H1.reshape(NC * N, F1 // 2),
                                               pk, z128)
    H2 = _mm_mid_kernel(N, NP, F1, F2, R, True)(S1, b1r, dinv, W2)
    S2 = _agg_kernel(N, NP, EP, F2 // 2, True)(H2.reshape(NC * N, F2 // 2),
                                               pk, z128)
    H3 = _mm_mid_kernel(N, NP, F2, F3, R, False)(S2, b2r, dinv, W3)
    S3 = _agg_kernel(N, NP, EP, F3, False)(H3, pk, z128)
    h, ls = _final_kernel(N, F3, R)(S3, b3r, dinv)
    return (h, ls)


# speedup vs baseline: 12.8140x; 1.1775x over previous
import functools

import jax
import jax.numpy as jnp
from jax import lax
from jax.experimental import pallas as pl
from jax.experimental.pallas import tpu as pltpu
from jax.experimental.pallas import tpu_sc as plsc

NC = 2
NS = 16
K = 128


def _zero_acc(z_hbm, acc, s, N, ZR):
    slab = (N // NS // 8) * 8
    last = N - slab * (NS - 1)

    @pl.when(s < NS - 1)
    def _():
        for i in range(slab // ZR):
            pltpu.sync_copy(z_hbm, acc.at[pl.ds(s * slab + i * ZR, ZR)])
        rem = slab % ZR
        if rem:
            pltpu.sync_copy(z_hbm.at[pl.ds(0, rem)],
                            acc.at[pl.ds(s * slab + (slab // ZR) * ZR, rem)])

    @pl.when(s == NS - 1)
    def _():
        base = slab * (NS - 1)
        for i in range(last // ZR):
            pltpu.sync_copy(z_hbm, acc.at[pl.ds(base + i * ZR, ZR)])
        rem = last % ZR
        if rem:
            pltpu.sync_copy(z_hbm.at[pl.ds(0, rem)],
                            acc.at[pl.ds(base + (last // ZR) * ZR, rem)])


def _copy_out(acc, out_hbm, c, s, N):
    slab = (N // NS // 8) * 8
    last = N - slab * (NS - 1)

    @pl.when(s < NS - 1)
    def _():
        pltpu.sync_copy(acc.at[pl.ds(s * slab, slab)],
                        out_hbm.at[c, pl.ds(s * slab, slab)])

    @pl.when(s == NS - 1)
    def _():
        base = slab * (NS - 1)
        pltpu.sync_copy(acc.at[pl.ds(base, last)],
                        out_hbm.at[c, pl.ds(base, last)])


def _deg_kernel(N, EP):
    T = EP // (NC * NS)
    G = T // K
    ZR = 128
    mesh = plsc.VectorSubcoreMesh(core_axis_name="c", subcore_axis_name="s")

    NB = 3
    assert G % NB == 0

    @functools.partial(
        pl.kernel,
        mesh=mesh,
        out_type=jax.ShapeDtypeStruct((NC, N, 128), jnp.float32),
        scratch_types=[
            pltpu.VMEM((NB, 4, K), jnp.int32),
            pltpu.VMEM((NB, K, 128), jnp.float32),
            pltpu.VMEM_SHARED((N, 128), jnp.float32),
            pltpu.SemaphoreType.DMA,
            pltpu.SemaphoreType.DMA,
            pltpu.SemaphoreType.DMA,
        ],
    )
    def deg_k(pk_hbm, z_hbm, out_hbm, pk, rows, acc, s0, s1, s2):
        c = lax.axis_index("c")
        s = lax.axis_index("s")
        ssem = (s0, s1, s2)

        for b in range(NB):
            pltpu.sync_copy(z_hbm, rows.at[b])
        _zero_acc(z_hbm, acc, s, N, ZR)
        plsc.subcore_barrier()

        cbase = (c * NS + s) * G

        def issue(ci, b):
            pltpu.sync_copy(pk_hbm.at[ci], pk.at[b])

        def process(b):
            def grp(g2, _):
                ei = pk[b, 3, pl.ds(g2 * 16, 16)]
                ews16 = lax.bitcast_convert_type(ei, jnp.float32)
                for l in range(16):
                    j = g2 * 16 + l
                    v = rows[b, j, pl.ds(0, 16)]
                    rows[b, j, pl.ds(0, 16)] = v * 0.0 + ews16[l]
                return 0
            lax.fori_loop(0, K // 16, grp, 0)
            pltpu.async_copy(rows.at[b], acc.at[pk.at[b].at[2]], ssem[b],
                             add=True)

        def wait_scatter(b):
            pltpu.make_async_copy(z_hbm, rows.at[b], ssem[b]).wait()

        issue(cbase, 0)
        G3 = G // NB

        def grp3(g3, _):
            g = g3 * NB
            for b in range(NB):
                b1 = (b + 1) % NB
                if b < NB - 1:
                    @pl.when(g3 > 0)
                    def _():
                        wait_scatter(b1)
                    issue(cbase + g + b + 1, b1)
                else:
                    @pl.when(g3 < G3 - 1)
                    def _():
                        wait_scatter(b1)
                        issue(cbase + g + b + 1, b1)
                process(b)
            return 0
        lax.fori_loop(0, G3, grp3, 0)
        for b in range(NB):
            wait_scatter(b)

        plsc.subcore_barrier()
        _copy_out(acc, out_hbm, c, s, N)

    return deg_k


def _agg_kernel(N, EP, Fh, split_features):
    T = EP // NS if split_features else EP // (NC * NS)
    G = T // K
    ZR = 128
    FB = Fh // 16
    mesh = plsc.VectorSubcoreMesh(core_axis_name="c", subcore_axis_name="s")

    NB = 3
    assert G % NB == 0

    @functools.partial(
        pl.kernel,
        mesh=mesh,
        out_type=jax.ShapeDtypeStruct((NC, N, Fh), jnp.float32),
        scratch_types=[
            pltpu.VMEM((NB, 4, K), jnp.int32),
            pltpu.VMEM((NB, K, Fh), jnp.float32),
            pltpu.VMEM_SHARED((N, Fh), jnp.float32),
            pltpu.SemaphoreType.DMA,
            pltpu.SemaphoreType.DMA,
            pltpu.SemaphoreType.DMA,
            pltpu.SemaphoreType.DMA,
            pltpu.SemaphoreType.DMA,
            pltpu.SemaphoreType.DMA,
        ],
    )
    def agg_k(h_hbm, pk_hbm, z_hbm, out_hbm, pk, rows, acc,
              g0, g1, g2s, t0, t1, t2):
        c = lax.axis_index("c")
        s = lax.axis_index("s")
        gsem = (g0, g1, g2s)
        ssem = (t0, t1, t2)

        _zero_acc(z_hbm, acc, s, N, ZR)
        plsc.subcore_barrier()

        if split_features:
            cbase = s * G
        else:
            cbase = (c * NS + s) * G

        def row_idx(b):
            if split_features:
                return pk.at[b].at[c]
            return pk.at[b].at[0]

        def issue(ci, b):
            pltpu.sync_copy(pk_hbm.at[ci], pk.at[b])
            pltpu.async_copy(h_hbm.at[row_idx(b)], rows.at[b], gsem[b])

        def process(b):
            pltpu.make_async_copy(h_hbm.at[pl.ds(0, K)], rows.at[b],
                                  gsem[b]).wait()

            def scale(g2, _):
                ei = pk[b, 3, pl.ds(g2 * 16, 16)]
                ews16 = lax.bitcast_convert_type(ei, jnp.float32)
                for l in range(16):
                    j = g2 * 16 + l
                    e = ews16[l]
                    for f in range(FB):
                        v = rows[b, j, pl.ds(f * 16, 16)]
                        rows[b, j, pl.ds(f * 16, 16)] = v * e
                return 0
            lax.fori_loop(0, K // 16, scale, 0)
            pltpu.async_copy(rows.at[b], acc.at[pk.at[b].at[2]], ssem[b],
                             add=True)

        def wait_scatter(b):
            pltpu.make_async_copy(h_hbm.at[pl.ds(0, K)], rows.at[b],
                                  ssem[b]).wait()

        issue(cbase, 0)
        G3 = G // NB

        def grp3(g3, _):
            g = g3 * NB
            for b in range(NB):
                b1 = (b + 1) % NB
                if b < NB - 1:
                    @pl.when(g3 > 0)
                    def _():
                        wait_scatter(b1)
                    issue(cbase + g + b + 1, b1)
                else:
                    @pl.when(g3 < G3 - 1)
                    def _():
                        wait_scatter(b1)
                        issue(cbase + g + b + 1, b1)
                process(b)
            return 0
        lax.fori_loop(0, G3, grp3, 0)
        for b in range(NB):
            wait_scatter(b)

        plsc.subcore_barrier()
        _copy_out(acc, out_hbm, c, s, N)

    return agg_k


_DOT = functools.partial(
    jax.lax.dot_general,
    dimension_numbers=(((1,), (0,)), ((), ())),
    precision=jax.lax.Precision.HIGHEST,
    preferred_element_type=jnp.float32,
)


def _dinv_kernel(N, R):
    def body(d_ref, out_ref):
        deg = d_ref[0, :, 0:1] + d_ref[1, :, 0:1]
        safe = jnp.where(deg > 0, deg, 1.0)
        out_ref[...] = jnp.where(deg > 0, lax.rsqrt(safe), 0.0)

    return pl.pallas_call(
        body,
        grid=(N // R,),
        in_specs=[pl.BlockSpec((NC, R, 128), lambda i: (0, i, 0))],
        out_specs=pl.BlockSpec((R, 1), lambda i: (i, 0)),
        out_shape=jax.ShapeDtypeStruct((N, 1), jnp.float32),
    )


def _mm_first_kernel(N, F_in, F_out, R):
    Fh = F_out // 2

    def body(x_ref, w_ref, dinv_ref, out_ref):
        h = _DOT(x_ref[...], w_ref[...]) * dinv_ref[...]
        out_ref[0] = h[:, :Fh]
        out_ref[1] = h[:, Fh:]

    return pl.pallas_call(
        body,
        grid=(N // R,),
        in_specs=[
            pl.BlockSpec((R, F_in), lambda i: (i, 0)),
            pl.BlockSpec((F_in, F_out), lambda i: (0, 0)),
            pl.BlockSpec((R, 1), lambda i: (i, 0)),
        ],
        out_specs=pl.BlockSpec((NC, R, Fh), lambda i: (0, i, 0)),
        out_shape=jax.ShapeDtypeStruct((NC, N, Fh), jnp.float32),
    )


def _mm_mid_kernel(N, F_in, F_out, R, split_out):
    Fih = F_in // 2
    Fh = F_out // 2

    def body(s_ref, b_ref, dinv_ref, w_ref, out_ref):
        dinv = dinv_ref[...]
        xa = jnp.maximum(s_ref[0] * dinv + b_ref[:, :Fih], 0.0)
        xb = jnp.maximum(s_ref[1] * dinv + b_ref[:, Fih:], 0.0)
        acc = _DOT(xa, w_ref[:Fih, :]) + _DOT(xb, w_ref[Fih:, :])
        h = acc * dinv
        if split_out:
            out_ref[0] = h[:, :Fh]
            out_ref[1] = h[:, Fh:]
        else:
            out_ref[...] = h

    if split_out:
        out_specs = pl.BlockSpec((NC, R, Fh), lambda i: (0, i, 0))
        out_shape = jax.ShapeDtypeStruct((NC, N, Fh), jnp.float32)
    else:
        out_specs = pl.BlockSpec((R, F_out), lambda i: (i, 0))
        out_shape = jax.ShapeDtypeStruct((N, F_out), jnp.float32)

    return pl.pallas_call(
        body,
        grid=(N // R,),
        in_specs=[
            pl.BlockSpec((NC, R, Fih), lambda i: (0, i, 0)),
            pl.BlockSpec((1, F_in), lambda i: (0, 0)),
            pl.BlockSpec((R, 1), lambda i: (i, 0)),
            pl.BlockSpec((F_in, F_out), lambda i: (0, 0)),
        ],
        out_specs=out_specs,
        out_shape=out_shape,
    )


def _final_kernel(N, F, R):
    def body(s_ref, b_ref, dinv_ref, h_ref, ls_ref):
        dinv = dinv_ref[...]
        z = (s_ref[0] + s_ref[1]) * dinv + b_ref[...]
        m = jnp.max(z, axis=1, keepdims=True)
        lse = jnp.log(jnp.sum(jnp.exp(z - m), axis=1, keepdims=True)) + m
        h_ref[...] = z
        ls_ref[...] = z - lse

    return pl.pallas_call(
        body,
        grid=(N // R,),
        in_specs=[
            pl.BlockSpec((NC, R, F), lambda i: (0, i, 0)),
            pl.BlockSpec((1, F), lambda i: (0, 0)),
            pl.BlockSpec((R, 1), lambda i: (i, 0)),
        ],
        out_specs=[
            pl.BlockSpec((R, F), lambda i: (i, 0)),
            pl.BlockSpec((R, F), lambda i: (i, 0)),
        ],
        out_shape=(
            jax.ShapeDtypeStruct((N, F), jnp.float32),
            jax.ShapeDtypeStruct((N, F), jnp.float32),
        ),
    )


def kernel(x, edge_index, edge_attr, W1, b1, W2, b2, W3, b3):
    N, F0 = x.shape
    E = edge_index.shape[1]
    F1, F2, F3 = W1.shape[1], W2.shape[1], W3.shape[1]
    R = 1000

    loop = jnp.arange(N, dtype=edge_index.dtype)
    row_e = jnp.concatenate([edge_index[0], loop])
    col_e = jnp.concatenate([edge_index[1], loop])
    ew_e = jnp.concatenate([edge_attr, jnp.ones((N,), x.dtype)])
    EE = E + N
    align = NC * NS * K * 3
    EP = ((EE + align - 1) // align) * align
    pad = EP - EE
    pidx = (jnp.arange(pad, dtype=edge_index.dtype) * 97) % N
    row_p = jnp.concatenate([row_e, pidx])
    col_p = jnp.concatenate([col_e, pidx])
    ew_p = jnp.concatenate([ew_e, jnp.zeros((pad,), x.dtype)])

    z128 = jnp.zeros((128, 128), jnp.float32)

    rowm = row_p.reshape(EP // K, K)
    colm = col_p.reshape(EP // K, K)
    ewb = jax.lax.bitcast_convert_type(ew_p, jnp.int32).reshape(EP // K, K)
    pk = jnp.stack([rowm, rowm + N, colm, ewb], axis=1)

    degacc = _deg_kernel(N, EP)(pk, z128)
    dinv = _dinv_kernel(N, R)(degacc)

    b1r = b1.reshape(1, F1)
    b2r = b2.reshape(1, F2)
    b3r = b3.reshape(1, F3)

    H1 = _mm_first_kernel(N, F0, F1, R)(x, W1, dinv)
    S1 = _agg_kernel(N, EP, F1 // 2, True)(H1.reshape(NC * N, F1 // 2),
                                           pk, z128)
    H2 = _mm_mid_kernel(N, F1, F2, R, True)(S1, b1r, dinv, W2)
    S2 = _agg_kernel(N, EP, F2 // 2, True)(H2.reshape(NC * N, F2 // 2),
                                           pk, z128)
    H3 = _mm_mid_kernel(N, F2, F3, R, False)(S2, b2r, dinv, W3)
    S3 = _agg_kernel(N, EP, F3, False)(H3, pk, z128)
    h, ls = _final_kernel(N, F3, R)(S3, b3r, dinv)
    return (h, ls)


# speedup vs baseline: 13.0190x; 1.0160x over previous
import functools

import jax
import jax.numpy as jnp
from jax import lax
from jax.experimental import pallas as pl
from jax.experimental.pallas import tpu as pltpu
from jax.experimental.pallas import tpu_sc as plsc

NC = 2
NS = 16
K = 128


def _zero_acc(z_hbm, acc, s, N, ZR):
    slab = (N // NS // 8) * 8
    last = N - slab * (NS - 1)

    @pl.when(s < NS - 1)
    def _():
        for i in range(slab // ZR):
            pltpu.sync_copy(z_hbm, acc.at[pl.ds(s * slab + i * ZR, ZR)])
        rem = slab % ZR
        if rem:
            pltpu.sync_copy(z_hbm.at[pl.ds(0, rem)],
                            acc.at[pl.ds(s * slab + (slab // ZR) * ZR, rem)])

    @pl.when(s == NS - 1)
    def _():
        base = slab * (NS - 1)
        for i in range(last // ZR):
            pltpu.sync_copy(z_hbm, acc.at[pl.ds(base + i * ZR, ZR)])
        rem = last % ZR
        if rem:
            pltpu.sync_copy(z_hbm.at[pl.ds(0, rem)],
                            acc.at[pl.ds(base + (last // ZR) * ZR, rem)])


def _copy_out(acc, out_hbm, c, s, N):
    slab = (N // NS // 8) * 8
    last = N - slab * (NS - 1)

    @pl.when(s < NS - 1)
    def _():
        pltpu.sync_copy(acc.at[pl.ds(s * slab, slab)],
                        out_hbm.at[c, pl.ds(s * slab, slab)])

    @pl.when(s == NS - 1)
    def _():
        base = slab * (NS - 1)
        pltpu.sync_copy(acc.at[pl.ds(base, last)],
                        out_hbm.at[c, pl.ds(base, last)])


def _deg_kernel(N, EP):
    T = EP // (NC * NS)
    G = T // K
    ZR = 128
    mesh = plsc.VectorSubcoreMesh(core_axis_name="c", subcore_axis_name="s")

    NB = 3
    assert G % NB == 0

    @functools.partial(
        pl.kernel,
        mesh=mesh,
        out_type=jax.ShapeDtypeStruct((NC, N, 128), jnp.float32),
        scratch_types=[
            pltpu.VMEM((NB, 4, K), jnp.int32),
            pltpu.VMEM((NB, K, 128), jnp.float32),
            pltpu.VMEM_SHARED((N, 128), jnp.float32),
            pltpu.SemaphoreType.DMA,
            pltpu.SemaphoreType.DMA,
            pltpu.SemaphoreType.DMA,
        ],
    )
    def deg_k(pk_hbm, z_hbm, out_hbm, pk, rows, acc, s0, s1, s2):
        c = lax.axis_index("c")
        s = lax.axis_index("s")
        ssem = (s0, s1, s2)

        for b in range(NB):
            pltpu.sync_copy(z_hbm, rows.at[b])
        _zero_acc(z_hbm, acc, s, N, ZR)
        plsc.subcore_barrier()

        cbase = (c * NS + s) * G

        def issue(ci, b):
            pltpu.sync_copy(pk_hbm.at[ci], pk.at[b])

        def process(b):
            def grp(g2, _):
                ei = pk[b, 3, pl.ds(g2 * 16, 16)]
                ews16 = lax.bitcast_convert_type(ei, jnp.float32)
                for l in range(16):
                    j = g2 * 16 + l
                    v = rows[b, j, pl.ds(0, 16)]
                    rows[b, j, pl.ds(0, 16)] = v * 0.0 + ews16[l]
                return 0
            lax.fori_loop(0, K // 16, grp, 0)
            pltpu.async_copy(rows.at[b], acc.at[pk.at[b].at[2]], ssem[b],
                             add=True)

        def wait_scatter(b):
            pltpu.make_async_copy(z_hbm, rows.at[b], ssem[b]).wait()

        issue(cbase, 0)
        G3 = G // NB

        def grp3(g3, _):
            g = g3 * NB
            for b in range(NB):
                b1 = (b + 1) % NB
                if b < NB - 1:
                    @pl.when(g3 > 0)
                    def _():
                        wait_scatter(b1)
                    issue(cbase + g + b + 1, b1)
                else:
                    @pl.when(g3 < G3 - 1)
                    def _():
                        wait_scatter(b1)
                        issue(cbase + g + b + 1, b1)
                process(b)
            return 0
        lax.fori_loop(0, G3, grp3, 0)
        for b in range(NB):
            wait_scatter(b)

        plsc.subcore_barrier()
        _copy_out(acc, out_hbm, c, s, N)

    return deg_k


def _agg_kernel(N, EP, Fh, split_features):
    T = EP // NS if split_features else EP // (NC * NS)
    G = T // K
    ZR = 128
    FB = Fh // 16
    mesh = plsc.VectorSubcoreMesh(core_axis_name="c", subcore_axis_name="s")

    NB = 3
    assert G % NB == 0

    @functools.partial(
        pl.kernel,
        mesh=mesh,
        out_type=jax.ShapeDtypeStruct((NC, N, Fh), jnp.float32),
        scratch_types=[
            pltpu.VMEM((NB, 4, K), jnp.int32),
            pltpu.VMEM((NB, K, Fh), jnp.float32),
            pltpu.VMEM_SHARED((N, Fh), jnp.float32),
            pltpu.SemaphoreType.DMA,
            pltpu.SemaphoreType.DMA,
            pltpu.SemaphoreType.DMA,
            pltpu.SemaphoreType.DMA,
            pltpu.SemaphoreType.DMA,
            pltpu.SemaphoreType.DMA,
        ],
    )
    def agg_k(h_hbm, pk_hbm, z_hbm, out_hbm, pk, rows, acc,
              g0, g1, g2s, t0, t1, t2):
        c = lax.axis_index("c")
        s = lax.axis_index("s")
        gsem = (g0, g1, g2s)
        ssem = (t0, t1, t2)

        _zero_acc(z_hbm, acc, s, N, ZR)
        plsc.subcore_barrier()

        if split_features:
            cbase = s * G
        else:
            cbase = (c * NS + s) * G

        def row_idx(b):
            if split_features:
                return pk.at[b].at[c]
            return pk.at[b].at[0]

        def issue(ci, b):
            pltpu.sync_copy(pk_hbm.at[ci], pk.at[b])
            pltpu.async_copy(h_hbm.at[row_idx(b)], rows.at[b], gsem[b])

        def process(b):
            pltpu.make_async_copy(h_hbm.at[pl.ds(0, K)], rows.at[b],
                                  gsem[b]).wait()

            def scale(g2, _):
                ei = pk[b, 3, pl.ds(g2 * 16, 16)]
                ews16 = lax.bitcast_convert_type(ei, jnp.float32)
                for l in range(16):
                    j = g2 * 16 + l
                    e = ews16[l]
                    for f in range(FB):
                        v = rows[b, j, pl.ds(f * 16, 16)]
                        rows[b, j, pl.ds(f * 16, 16)] = v * e
                return 0
            lax.fori_loop(0, K // 16, scale, 0)
            pltpu.async_copy(rows.at[b], acc.at[pk.at[b].at[2]], ssem[b],
                             add=True)

        def wait_scatter(b):
            pltpu.make_async_copy(h_hbm.at[pl.ds(0, K)], rows.at[b],
                                  ssem[b]).wait()

        issue(cbase, 0)
        G3 = G // NB

        def grp3(g3, _):
            g = g3 * NB
            for b in range(NB):
                b1 = (b + 1) % NB
                if b < NB - 1:
                    @pl.when(g3 > 0)
                    def _():
                        wait_scatter(b1)
                    issue(cbase + g + b + 1, b1)
                else:
                    @pl.when(g3 < G3 - 1)
                    def _():
                        wait_scatter(b1)
                        issue(cbase + g + b + 1, b1)
                process(b)
            return 0
        lax.fori_loop(0, G3, grp3, 0)
        for b in range(NB):
            wait_scatter(b)

        plsc.subcore_barrier()
        _copy_out(acc, out_hbm, c, s, N)

    return agg_k


_DOT = functools.partial(
    jax.lax.dot_general,
    dimension_numbers=(((1,), (0,)), ((), ())),
    precision=jax.lax.Precision.HIGHEST,
    preferred_element_type=jnp.float32,
)


def _mm_first_kernel(N, F_in, F_out, R):
    Fh = F_out // 2

    def body(x_ref, w_ref, d_ref, out_ref, dinv_ref):
        deg = d_ref[0, :, 0:1] + d_ref[1, :, 0:1]
        safe = jnp.where(deg > 0, deg, 1.0)
        dinv = jnp.where(deg > 0, lax.rsqrt(safe), 0.0)
        dinv_ref[...] = dinv
        h = _DOT(x_ref[...], w_ref[...]) * dinv
        out_ref[0] = h[:, :Fh]
        out_ref[1] = h[:, Fh:]

    return pl.pallas_call(
        body,
        grid=(N // R,),
        in_specs=[
            pl.BlockSpec((R, F_in), lambda i: (i, 0)),
            pl.BlockSpec((F_in, F_out), lambda i: (0, 0)),
            pl.BlockSpec((NC, R, 128), lambda i: (0, i, 0)),
        ],
        out_specs=[
            pl.BlockSpec((NC, R, Fh), lambda i: (0, i, 0)),
            pl.BlockSpec((R, 1), lambda i: (i, 0)),
        ],
        out_shape=(
            jax.ShapeDtypeStruct((NC, N, Fh), jnp.float32),
            jax.ShapeDtypeStruct((N, 1), jnp.float32),
        ),
    )


def _mm_mid_kernel(N, F_in, F_out, R, split_out):
    Fih = F_in // 2
    Fh = F_out // 2

    def body(s_ref, b_ref, dinv_ref, w_ref, out_ref):
        dinv = dinv_ref[...]
        xa = jnp.maximum(s_ref[0] * dinv + b_ref[:, :Fih], 0.0)
        xb = jnp.maximum(s_ref[1] * dinv + b_ref[:, Fih:], 0.0)
        acc = _DOT(xa, w_ref[:Fih, :]) + _DOT(xb, w_ref[Fih:, :])
        h = acc * dinv
        if split_out:
            out_ref[0] = h[:, :Fh]
            out_ref[1] = h[:, Fh:]
        else:
            out_ref[...] = h

    if split_out:
        out_specs = pl.BlockSpec((NC, R, Fh), lambda i: (0, i, 0))
        out_shape = jax.ShapeDtypeStruct((NC, N, Fh), jnp.float32)
    else:
        out_specs = pl.BlockSpec((R, F_out), lambda i: (i, 0))
        out_shape = jax.ShapeDtypeStruct((N, F_out), jnp.float32)

    return pl.pallas_call(
        body,
        grid=(N // R,),
        in_specs=[
            pl.BlockSpec((NC, R, Fih), lambda i: (0, i, 0)),
            pl.BlockSpec((1, F_in), lambda i: (0, 0)),
            pl.BlockSpec((R, 1), lambda i: (i, 0)),
            pl.BlockSpec((F_in, F_out), lambda i: (0, 0)),
        ],
        out_specs=out_specs,
        out_shape=out_shape,
    )


def _final_kernel(N, F, R):
    def body(s_ref, b_ref, dinv_ref, h_ref, ls_ref):
        dinv = dinv_ref[...]
        z = (s_ref[0] + s_ref[1]) * dinv + b_ref[...]
        m = jnp.max(z, axis=1, keepdims=True)
        lse = jnp.log(jnp.sum(jnp.exp(z - m), axis=1, keepdims=True)) + m
        h_ref[...] = z
        ls_ref[...] = z - lse

    return pl.pallas_call(
        body,
        grid=(N // R,),
        in_specs=[
            pl.BlockSpec((NC, R, F), lambda i: (0, i, 0)),
            pl.BlockSpec((1, F), lambda i: (0, 0)),
            pl.BlockSpec((R, 1), lambda i: (i, 0)),
        ],
        out_specs=[
            pl.BlockSpec((R, F), lambda i: (i, 0)),
            pl.BlockSpec((R, F), lambda i: (i, 0)),
        ],
        out_shape=(
            jax.ShapeDtypeStruct((N, F), jnp.float32),
            jax.ShapeDtypeStruct((N, F), jnp.float32),
        ),
    )


def kernel(x, edge_index, edge_attr, W1, b1, W2, b2, W3, b3):
    N, F0 = x.shape
    E = edge_index.shape[1]
    F1, F2, F3 = W1.shape[1], W2.shape[1], W3.shape[1]
    R = 1000

    loop = jnp.arange(N, dtype=edge_index.dtype)
    row_e = jnp.concatenate([edge_index[0], loop])
    col_e = jnp.concatenate([edge_index[1], loop])
    ew_e = jnp.concatenate([edge_attr, jnp.ones((N,), x.dtype)])
    EE = E + N
    align = NC * NS * K * 3
    EP = ((EE + align - 1) // align) * align
    pad = EP - EE
    pidx = (jnp.arange(pad, dtype=edge_index.dtype) * 97) % N
    row_p = jnp.concatenate([row_e, pidx])
    col_p = jnp.concatenate([col_e, pidx])
    ew_p = jnp.concatenate([ew_e, jnp.zeros((pad,), x.dtype)])

    z128 = jnp.zeros((128, 128), jnp.float32)

    rowm = row_p.reshape(EP // K, K)
    colm = col_p.reshape(EP // K, K)
    ewb = jax.lax.bitcast_convert_type(ew_p, jnp.int32).reshape(EP // K, K)
    pk = jnp.stack([rowm, rowm + N, colm, ewb], axis=1)

    degacc = _deg_kernel(N, EP)(pk, z128)

    b1r = b1.reshape(1, F1)
    b2r = b2.reshape(1, F2)
    b3r = b3.reshape(1, F3)

    H1, dinv = _mm_first_kernel(N, F0, F1, R)(x, W1, degacc)
    S1 = _agg_kernel(N, EP, F1 // 2, True)(H1.reshape(NC * N, F1 // 2),
                                           pk, z128)
    H2 = _mm_mid_kernel(N, F1, F2, R, True)(S1, b1r, dinv, W2)
    S2 = _agg_kernel(N, EP, F2 // 2, True)(H2.reshape(NC * N, F2 // 2),
                                           pk, z128)
    H3 = _mm_mid_kernel(N, F2, F3, R, False)(S2, b2r, dinv, W3)
    S3 = _agg_kernel(N, EP, F3, False)(H3, pk, z128)
    h, ls = _final_kernel(N, F3, R)(S3, b3r, dinv)
    return (h, ls)
